# baseline (speedup 1.0000x reference)
import jax
import jax.numpy as jnp
from jax import lax
from jax.experimental import pallas as pl
from jax.experimental.pallas import tpu as pltpu
from jax.experimental.pallas import tpu_sc as plsc

VOCAB = 100000
EMBED = 64
MAX_LEN = 200
BATCH = 4096

NUM_CORES = 2
NUM_SUBCORES = 16
NW = NUM_CORES * NUM_SUBCORES
LANES = 16

ROWS_PER_W = BATCH // NW
IDX_HALF = ROWS_PER_W // 2
SPLIT = 128
REST = MAX_LEN - SPLIT
NBUF = 4
NROUNDS = ROWS_PER_W // NBUF
RESTAGE_ROUND = IDX_HALF // NBUF - 1

HALF = MAX_LEN // 2
P_CHUNKS = 128 // LANES


def _body(idx_hbm, table_hbm, pos_hbm, out_hbm, idx_v, pos_v, ins, outs,
          gsem, osem):
    wid = lax.axis_index("s") * NUM_CORES + lax.axis_index("c")
    row0 = wid * ROWS_PER_W

    pltpu.sync_copy(idx_hbm.at[pl.ds(row0, IDX_HALF)], idx_v)
    pltpu.sync_copy(pos_hbm, pos_v)

    def gather_copies(b, r):
        i = lax.rem(r, IDX_HALF)
        return (
            (table_hbm.at[idx_v.at[i, pl.ds(0, SPLIT)]],
             ins[b].at[pl.ds(0, SPLIT)]),
            (table_hbm.at[idx_v.at[i, pl.ds(SPLIT, REST)]],
             ins[b].at[pl.ds(SPLIT, REST)]),
        )

    def issue_gathers(b, r):
        for src, dst in gather_copies(b, r):
            pltpu.async_copy(src, dst, gsem[b])

    def wait_gathers(b, r):
        for src, dst in gather_copies(b, r):
            pltpu.make_async_copy(src, dst, gsem[b]).wait()

    def out_dst(r):
        g = row0 + r
        return out_hbm.at[g // 8, :, lax.rem(g, 8)]

    for b in range(NBUF):
        issue_gathers(b, b)

    @pl.loop(0, NROUNDS)
    def _round(rnd):
        base = rnd * NBUF
        for b in range(NBUF):
            r = base + b
            wait_gathers(b, r)

            @pl.when(r >= NBUF)
            def _():
                pltpu.make_async_copy(outs[b], out_dst(r - NBUF),
                                      osem[b]).wait()

            @pl.loop(0, HALF, unroll=4)
            def _pos(k):
                for c in range(P_CHUNKS):
                    x = ins[b][2 * k + c // 4, pl.ds(16 * (c % 4), LANES)]
                    p = pos_v[k, pl.ds(c * LANES, LANES)]
                    outs[b][k, pl.ds(c * LANES, LANES)] = x + p

            pltpu.async_copy(outs[b], out_dst(r), osem[b])

        @pl.when(rnd == RESTAGE_ROUND)
        def _restage():
            pltpu.sync_copy(idx_hbm.at[pl.ds(row0 + IDX_HALF, IDX_HALF)],
                            idx_v)

        @pl.when(rnd < NROUNDS - 1)
        def _refill():
            for b in range(NBUF):
                issue_gathers(b, base + b + NBUF)

    for b in range(NBUF):
        pltpu.make_async_copy(outs[b], out_dst(ROWS_PER_W - NBUF + b),
                              osem[b]).wait()


@jax.jit
def _embed(idx, token_table, pos2):
    mesh = plsc.VectorSubcoreMesh(
        core_axis_name="c", subcore_axis_name="s",
        num_cores=NUM_CORES, num_subcores=NUM_SUBCORES)
    mid = pl.kernel(
        _body,
        out_type=jax.ShapeDtypeStruct((BATCH // 8, HALF, 8, 128), jnp.float32),
        mesh=mesh,
        compiler_params=pltpu.CompilerParams(use_tc_tiling_on_sc=False),
        scratch_types=[
            pltpu.VMEM((IDX_HALF, MAX_LEN), jnp.int32),
            pltpu.VMEM((HALF, 128), jnp.float32),
            [pltpu.VMEM((MAX_LEN, EMBED), jnp.float32) for _ in range(NBUF)],
            [pltpu.VMEM((HALF, 128), jnp.float32) for _ in range(NBUF)],
            [pltpu.SemaphoreType.DMA for _ in range(NBUF)],
            [pltpu.SemaphoreType.DMA for _ in range(NBUF)],
        ],
    )(idx, token_table, pos2)
    mid2 = mid.transpose(0, 2, 1, 3).reshape(BATCH, MAX_LEN * EMBED)
    out2 = mid2.T
    return jnp.transpose(out2.reshape(MAX_LEN, EMBED, BATCH), (2, 0, 1))


def kernel(input_seq, token_table, pos_table):
    return _embed(input_seq.astype(jnp.int32), token_table,
                  pos_table.reshape(HALF, 128))


# speedup vs baseline: 1.1764x; 1.1764x over previous
import jax
import jax.numpy as jnp
from jax import lax
from jax.experimental import pallas as pl
from jax.experimental.pallas import tpu as pltpu
from jax.experimental.pallas import tpu_sc as plsc

VOCAB = 100000
EMBED = 64
MAX_LEN = 200
BATCH = 4096

NUM_CORES = 2
NUM_SUBCORES = 16
NW = NUM_CORES * NUM_SUBCORES
LANES = 16

ROWS_PER_W = BATCH // NW
NGRP = ROWS_PER_W // 8
NSLC = 5
LSLC = MAX_LEN // NSLC
JSLC = LSLC * EMBED // 128
UNITS = NGRP * NSLC
HALF = MAX_LEN // 2
P_CHUNKS = 128 // LANES


def _body(idx_hbm, table_hbm, pos_hbm, out_hbm, idx_v, pos_v, ins, outs,
          gsem, osem):
    wid = lax.axis_index("s") * NUM_CORES + lax.axis_index("c")
    row0 = wid * ROWS_PER_W
    tile0 = wid * NGRP

    pltpu.sync_copy(idx_hbm.at[pl.ds(row0, ROWS_PER_W)], idx_v)
    pltpu.sync_copy(pos_hbm, pos_v)

    def unit_gathers(p, u):
        g8 = u // NSLC
        q = lax.rem(u, NSLC)
        return [
            (table_hbm.at[idx_v.at[g8 * 8 + s, pl.ds(q * LSLC, LSLC)]],
             ins[p].at[s])
            for s in range(8)
        ]

    def issue_gathers(p, u):
        for src, dst in unit_gathers(p, u):
            pltpu.async_copy(src, dst, gsem[p])

    def wait_gathers(p, u):
        for src, dst in unit_gathers(p, u):
            pltpu.make_async_copy(src, dst, gsem[p]).wait()

    def out_dst(u):
        g8 = u // NSLC
        q = lax.rem(u, NSLC)
        return out_hbm.at[tile0 + g8, pl.ds(q * JSLC, JSLC)]

    issue_gathers(0, 0)
    issue_gathers(1, 1)

    @pl.loop(0, UNITS // 2)
    def _round(rnd):
        for p in range(2):
            u = rnd * 2 + p
            wait_gathers(p, u)

            @pl.when(u >= 2)
            def _():
                pltpu.make_async_copy(outs[p], out_dst(u - 2), osem[p]).wait()

            q = lax.rem(u, NSLC)
            j0 = q * JSLC

            @pl.loop(0, JSLC, unroll=2)
            def _pos(k):
                for c in range(P_CHUNKS):
                    pv = pos_v[j0 + k, pl.ds(c * LANES, LANES)]
                    lrow = 2 * k + c // 4
                    doff = 16 * (c % 4)
                    for s in range(8):
                        x = ins[p][s, lrow, pl.ds(doff, LANES)]
                        outs[p][k, s, pl.ds(c * LANES, LANES)] = x + pv

            pltpu.async_copy(outs[p], out_dst(u), osem[p])

            @pl.when(u + 2 < UNITS)
            def _():
                issue_gathers(p, u + 2)

    for p in range(2):
        pltpu.make_async_copy(outs[p], out_dst(UNITS - 2 + p), osem[p]).wait()


@jax.jit
def _embed(idx, token_table, pos2):
    mesh = plsc.VectorSubcoreMesh(
        core_axis_name="c", subcore_axis_name="s",
        num_cores=NUM_CORES, num_subcores=NUM_SUBCORES)
    mid = pl.kernel(
        _body,
        out_type=jax.ShapeDtypeStruct((BATCH // 8, HALF, 8, 128), jnp.float32),
        mesh=mesh,
        compiler_params=pltpu.CompilerParams(use_tc_tiling_on_sc=False),
        scratch_types=[
            pltpu.VMEM((ROWS_PER_W, MAX_LEN), jnp.int32),
            pltpu.VMEM((HALF, 128), jnp.float32),
            [pltpu.VMEM((8, LSLC, EMBED), jnp.float32) for _ in range(2)],
            [pltpu.VMEM((JSLC, 8, 128), jnp.float32) for _ in range(2)],
            [pltpu.SemaphoreType.DMA for _ in range(2)],
            [pltpu.SemaphoreType.DMA for _ in range(2)],
        ],
    )(idx, token_table, pos2)
    mid2 = mid.transpose(0, 2, 1, 3).reshape(BATCH, MAX_LEN * EMBED)
    out2 = mid2.T
    return jnp.transpose(out2.reshape(MAX_LEN, EMBED, BATCH), (2, 0, 1))


def kernel(input_seq, token_table, pos_table):
    return _embed(input_seq.astype(jnp.int32), token_table,
                  pos_table.reshape(HALF, 128))


# speedup vs baseline: 1.4321x; 1.2174x over previous
import jax
import jax.numpy as jnp
from jax import lax
from jax.experimental import pallas as pl
from jax.experimental.pallas import tpu as pltpu
from jax.experimental.pallas import tpu_sc as plsc

VOCAB = 100000
EMBED = 64
MAX_LEN = 200
BATCH = 4096

NUM_CORES = 2
NUM_SUBCORES = 16
NW = NUM_CORES * NUM_SUBCORES
LANES = 16

ROWS_PER_W = BATCH // NW
NGRP = ROWS_PER_W // 8
NSLC = 5
LSLC = MAX_LEN // NSLC
USIZE = 8 * LSLC
UNITS = NGRP * NSLC
HALF = MAX_LEN // 2
NBUF = 4


def _body(idx_hbm, table_hbm, pos_hbm, out_hbm, idx_v, pos_v, bufs,
          gsem, osem):
    wid = lax.axis_index("s") * NUM_CORES + lax.axis_index("c")
    u0 = wid * UNITS

    pltpu.sync_copy(idx_hbm.at[pl.ds(u0, UNITS)], idx_v)
    pltpu.sync_copy(pos_hbm, pos_v)

    def unit_gathers(b, u):
        return [
            (table_hbm.at[idx_v.at[u, pl.ds(o, n)]],
             bufs[b].at[pl.ds(o, n)])
            for o, n in ((0, 128), (128, 128), (256, 64))
        ]

    def issue_gathers(b, u):
        for src, dst in unit_gathers(b, u):
            pltpu.async_copy(src, dst, gsem[b])

    def wait_gathers(b, u):
        for src, dst in unit_gathers(b, u):
            pltpu.make_async_copy(src, dst, gsem[b]).wait()

    def out_dst(u):
        return out_hbm.at[pl.ds((u0 + u) * USIZE, USIZE)]

    for b in range(NBUF):
        issue_gathers(b, b)

    @pl.loop(0, UNITS // NBUF)
    def _round(rnd):
        for j in range(NBUF):
            u = rnd * NBUF + j
            wait_gathers(j, u)

            q = lax.rem(u, NSLC)
            j0 = q * (LSLC // 2)

            @pl.loop(0, LSLC // 2, unroll=2)
            def _pos(k):
                for h in range(2):
                    for c4 in range(4):
                        pv = pos_v[j0 + k, pl.ds(16 * (4 * h + c4), LANES)]
                        for s in range(8):
                            plsc.addupdate(
                                bufs[j].at[16 * k + 2 * s + h,
                                           pl.ds(16 * c4, LANES)], pv)

            pltpu.async_copy(bufs[j], out_dst(u), osem[j])

            jp = (j - 1) % NBUF

            @pl.when(u >= 1)
            def _():
                pltpu.make_async_copy(bufs[jp], out_dst(u - 1), osem[jp]).wait()

                @pl.when(u + NBUF - 1 < UNITS)
                def _():
                    issue_gathers(jp, u + NBUF - 1)

    pltpu.make_async_copy(bufs[(UNITS - 1) % NBUF], out_dst(UNITS - 1),
                          osem[(UNITS - 1) % NBUF]).wait()


@jax.jit
def _embed(idx_p, token_table, pos2):
    mesh = plsc.VectorSubcoreMesh(
        core_axis_name="c", subcore_axis_name="s",
        num_cores=NUM_CORES, num_subcores=NUM_SUBCORES)
    mid = pl.kernel(
        _body,
        out_type=jax.ShapeDtypeStruct((BATCH * MAX_LEN, EMBED), jnp.float32),
        mesh=mesh,
        compiler_params=pltpu.CompilerParams(use_tc_tiling_on_sc=False),
        scratch_types=[
            pltpu.VMEM((UNITS, USIZE), jnp.int32),
            pltpu.VMEM((HALF, 128), jnp.float32),
            [pltpu.VMEM((USIZE, EMBED), jnp.float32) for _ in range(NBUF)],
            [pltpu.SemaphoreType.DMA for _ in range(NBUF)],
            [pltpu.SemaphoreType.DMA for _ in range(NBUF)],
        ],
    )(idx_p, token_table, pos2)
    mid2 = (mid.reshape(BATCH // 8, HALF, 8, 128)
            .transpose(0, 2, 1, 3).reshape(BATCH, MAX_LEN * EMBED))
    out2 = mid2.T
    return jnp.transpose(out2.reshape(MAX_LEN, EMBED, BATCH), (2, 0, 1))


def kernel(input_seq, token_table, pos_table):
    idx_p = (input_seq.astype(jnp.int32)
             .reshape(NW * NGRP, 8, NSLC, LSLC // 2, 2)
             .transpose(0, 2, 3, 1, 4)
             .reshape(NW * UNITS, USIZE))
    return _embed(idx_p, token_table, pos_table.reshape(HALF, 128))


# speedup vs baseline: 2.2101x; 1.5432x over previous
import jax
import jax.numpy as jnp
from jax import lax
from jax.experimental import pallas as pl
from jax.experimental.pallas import tpu as pltpu
from jax.experimental.pallas import tpu_sc as plsc

VOCAB = 100000
EMBED = 64
MAX_LEN = 200
BATCH = 4096

NUM_CORES = 2
NUM_SUBCORES = 16
NW = NUM_CORES * NUM_SUBCORES
LANES = 16

ROWS_PER_W = BATCH // NW
NGRP = ROWS_PER_W // 8
NSLC = 5
LSLC = MAX_LEN // NSLC
USIZE = 8 * LSLC
UNITS = NGRP * NSLC
HALF = MAX_LEN // 2
NBUF = 4


def _body(idx_hbm, table_hbm, pos_hbm, out_hbm, idx_v, pos_v, bufs, idxp,
          gsem, osem):
    wid = lax.axis_index("s") * NUM_CORES + lax.axis_index("c")
    row0 = wid * ROWS_PER_W
    u0 = wid * UNITS

    pltpu.sync_copy(idx_hbm.at[pl.ds(row0, ROWS_PER_W)], idx_v)
    pltpu.sync_copy(pos_hbm, pos_v)

    iota = lax.iota(jnp.int32, LANES)
    lane_s = lax.shift_right_logical(iota, 1)
    lane_h = lax.bitwise_and(iota, 1)

    def unit_gathers(b, u):
        return [
            (table_hbm.at[idxp[b].at[pl.ds(o, n)]],
             bufs[b].at[pl.ds(o, n)])
            for o, n in ((0, 128), (128, 128), (256, 64))
        ]

    def issue_gathers(b, u):
        g8 = (u // NSLC) * 8
        q = lax.rem(u, NSLC)
        row_v = g8 + lane_s
        col0 = q * LSLC + lane_h
        for k in range(LSLC // 2):
            vals = plsc.load_gather(idx_v, [row_v, col0 + 2 * k])
            idxp[b][pl.ds(LANES * k, LANES)] = vals
        for src, dst in unit_gathers(b, u):
            pltpu.async_copy(src, dst, gsem[b])

    def wait_gathers(b, u):
        for src, dst in unit_gathers(b, u):
            pltpu.make_async_copy(src, dst, gsem[b]).wait()

    def out_dst(u):
        return out_hbm.at[pl.ds((u0 + u) * USIZE, USIZE)]

    for b in range(NBUF):
        issue_gathers(b, b)

    @pl.loop(0, UNITS // NBUF)
    def _round(rnd):
        for j in range(NBUF):
            u = rnd * NBUF + j
            wait_gathers(j, u)

            q = lax.rem(u, NSLC)
            j0 = q * (LSLC // 2)

            @pl.loop(0, LSLC // 2, unroll=2)
            def _pos(k):
                for h in range(2):
                    for c4 in range(4):
                        pv = pos_v[j0 + k, pl.ds(16 * (4 * h + c4), LANES)]
                        for s in range(8):
                            plsc.addupdate(
                                bufs[j].at[16 * k + 2 * s + h,
                                           pl.ds(16 * c4, LANES)], pv)

            pltpu.async_copy(bufs[j], out_dst(u), osem[j])

            jp = (j - 1) % NBUF

            @pl.when(u >= 1)
            def _():
                pltpu.make_async_copy(bufs[jp], out_dst(u - 1), osem[jp]).wait()

                @pl.when(u + NBUF - 1 < UNITS)
                def _():
                    issue_gathers(jp, u + NBUF - 1)

    pltpu.make_async_copy(bufs[(UNITS - 1) % NBUF], out_dst(UNITS - 1),
                          osem[(UNITS - 1) % NBUF]).wait()


@jax.jit
def _embed(idx, token_table, pos2):
    mesh = plsc.VectorSubcoreMesh(
        core_axis_name="c", subcore_axis_name="s",
        num_cores=NUM_CORES, num_subcores=NUM_SUBCORES)
    mid = pl.kernel(
        _body,
        out_type=jax.ShapeDtypeStruct((BATCH * MAX_LEN, EMBED), jnp.float32),
        mesh=mesh,
        compiler_params=pltpu.CompilerParams(use_tc_tiling_on_sc=False,
                                             needs_layout_passes=False),
        scratch_types=[
            pltpu.VMEM((ROWS_PER_W, MAX_LEN), jnp.int32),
            pltpu.VMEM((HALF, 128), jnp.float32),
            [pltpu.VMEM((USIZE, EMBED), jnp.float32) for _ in range(NBUF)],
            [pltpu.VMEM((USIZE,), jnp.int32) for _ in range(NBUF)],
            [pltpu.SemaphoreType.DMA for _ in range(NBUF)],
            [pltpu.SemaphoreType.DMA for _ in range(NBUF)],
        ],
    )(idx, token_table, pos2)
    mid2 = (mid.reshape(BATCH // 8, HALF, 8, 128)
            .transpose(0, 2, 1, 3).reshape(BATCH, MAX_LEN * EMBED))
    out2 = mid2.T
    return jnp.transpose(out2.reshape(MAX_LEN, EMBED, BATCH), (2, 0, 1))


def kernel(input_seq, token_table, pos_table):
    return _embed(input_seq.astype(jnp.int32), token_table,
                  pos_table.reshape(HALF, 128))
